# Initial kernel scaffold; baseline (speedup 1.0000x reference)
import functools

import jax
import jax.numpy as jnp
from jax import lax
from jax.experimental import pallas as pl
from jax.experimental.pallas import tpu as pltpu
from jax.experimental.pallas import tpu_sc as plsc

_N = 2048
_D = 1024
_H = 2048
_E = 8
_BN = 128
_T = 40
_R = _T * _BN
_PAIRS = 2 * _N
_NW = 32
_CH = 64
_NCH = _PAIRS // (_NW * _CH)


def _router_body(x_ref, wg_ref, pos_ref, w_ref, te_ref, nact_ref, bl_ref):
    xf = x_ref[...]
    wg = wg_ref[...]
    logits = lax.dot_general(xf, wg, (((1,), (0,)), ((), ())),
                             preferred_element_type=jnp.float32)

    u8 = (lax.broadcasted_iota(jnp.int32, (_E, _E), 0)
          < lax.broadcasted_iota(jnp.int32, (_E, _E), 1)).astype(jnp.float32)

    def first_max_mask(v):
        m = jnp.max(v, axis=1, keepdims=True)
        ism = (v == m).astype(jnp.float32)
        prev = lax.dot_general(ism, u8, (((1,), (0,)), ((), ())),
                               preferred_element_type=jnp.float32)
        return (ism > 0) & (prev == 0.0), m

    oh1b, m1 = first_max_mask(logits)
    masked = jnp.where(oh1b, -1e30, logits)
    oh2b, m2 = first_max_mask(masked)
    s1 = jax.nn.sigmoid(m1 - m2)
    s2 = jax.nn.sigmoid(m2 - m1)
    oh1 = oh1b.astype(jnp.float32)
    oh2 = oh2b.astype(jnp.float32)

    imp = jnp.sum(oh1 * s1 + oh2 * s2, axis=0, keepdims=True)
    load = jnp.sum(oh1 + oh2, axis=0, keepdims=True)

    def cv2(v):
        mean = jnp.sum(v) / _E
        var = jnp.sum((v - mean) ** 2) / (_E - 1)
        return var / (mean * mean + 1e-10)

    bl_ref[0, 0] = cv2(imp) * cv2(load) * 0.01

    ls = (lax.broadcasted_iota(jnp.int32, (_BN, _BN), 0)
          > lax.broadcasted_iota(jnp.int32, (_BN, _BN), 1)).astype(jnp.float32)

    def excl_cumsum(oh):
        chunks = []
        carry = jnp.zeros((1, _E), jnp.float32)
        for c in range(_N // _BN):
            blk = oh[c * _BN:(c + 1) * _BN, :]
            cex = lax.dot_general(ls, blk, (((1,), (0,)), ((), ())),
                                  preferred_element_type=jnp.float32) + carry
            chunks.append(cex)
            carry = carry + jnp.sum(blk, axis=0, keepdims=True)
        return jnp.concatenate(chunks, axis=0), carry

    c1, tot1 = excl_cumsum(oh1)
    c2, tot2 = excl_cumsum(oh2)
    counts = tot1 + tot2
    pcq = (counts.astype(jnp.int32) + (_BN - 1)) // _BN
    baseq = lax.dot_general(pcq.astype(jnp.float32), u8,
                            (((1,), (0,)), ((), ())),
                            preferred_element_type=jnp.float32).astype(jnp.int32)
    ends = baseq + pcq
    nact_ref[0, 0] = jnp.sum(pcq)

    t_iota = lax.broadcasted_iota(jnp.int32, (_T, _E), 0)
    te_raw = jnp.sum((t_iota >= jnp.broadcast_to(ends, (_T, _E)))
                     .astype(jnp.int32), axis=1)
    max_e = jnp.max(jnp.where(pcq > 0,
                              lax.broadcasted_iota(jnp.int32, (1, _E), 1), -1))
    te_ref[...] = jnp.minimum(te_raw, max_e).reshape(1, _T)

    basef = (baseq * _BN).astype(jnp.float32)
    base_b = jnp.broadcast_to(basef, (_N, _E))
    rank0 = jnp.sum(c1 * oh1, axis=1, keepdims=True)
    rank1 = jnp.sum((jnp.broadcast_to(tot1, (_N, _E)) + c2) * oh2,
                    axis=1, keepdims=True)
    base0 = jnp.sum(base_b * oh1, axis=1, keepdims=True)
    base1 = jnp.sum(base_b * oh2, axis=1, keepdims=True)
    pos0 = (base0 + rank0).astype(jnp.int32)
    pos1 = (base1 + rank1).astype(jnp.int32)
    pos_ref[...] = jnp.concatenate([pos0, pos1], axis=1)
    w_ref[...] = jnp.concatenate([s1, s2], axis=1)


_sc_mesh = plsc.VectorSubcoreMesh(core_axis_name="c", subcore_axis_name="s")


@functools.partial(
    pl.kernel, mesh=_sc_mesh,
    out_type=jax.ShapeDtypeStruct((_R, _D), jnp.float32),
    scratch_types=[pltpu.VMEM((_CH,), jnp.int32),
                   pltpu.VMEM((_CH, _D), jnp.float32),
                   pltpu.SemaphoreType.DMA])
def _sc_scatter(x_hbm, pos_hbm, xg_hbm, idx_v, rows_v, sem):
    wid = lax.axis_index("s") * 2 + lax.axis_index("c")
    for c in range(_NCH):
        base = wid * (_CH * _NCH) + c * _CH
        pltpu.sync_copy(pos_hbm.at[pl.ds(base, _CH)], idx_v)
        src = lax.rem(base, _N)
        pltpu.sync_copy(x_hbm.at[pl.ds(src, _CH)], rows_v)
        pltpu.async_copy(rows_v, xg_hbm.at[idx_v], sem).wait()


@functools.partial(
    pl.kernel, mesh=_sc_mesh,
    out_type=jax.ShapeDtypeStruct((_PAIRS, _D), jnp.float32),
    scratch_types=[pltpu.VMEM((_CH,), jnp.int32),
                   pltpu.VMEM((_CH, _D), jnp.float32),
                   pltpu.SemaphoreType.DMA])
def _sc_gather(yg_hbm, pos_hbm, yc_hbm, idx_v, rows_v, sem):
    wid = lax.axis_index("s") * 2 + lax.axis_index("c")
    for c in range(_NCH):
        base = wid * (_CH * _NCH) + c * _CH
        pltpu.sync_copy(pos_hbm.at[pl.ds(base, _CH)], idx_v)
        pltpu.async_copy(yg_hbm.at[idx_v], rows_v, sem).wait()
        pltpu.sync_copy(rows_v, yc_hbm.at[pl.ds(base, _CH)])


def _expert_body(te_ref, na_ref, xg_ref, wg_ref, wu_ref, wd_ref,
                 bg_ref, bu_ref, bd_ref, out_ref):
    t = pl.program_id(0)

    @pl.when(t < na_ref[0, 0])
    def _():
        xb = xg_ref[...]
        g = lax.dot_general(xb, wg_ref[0], (((1,), (1,)), ((), ())),
                            preferred_element_type=jnp.float32) + bg_ref[...]
        u = lax.dot_general(xb, wu_ref[0], (((1,), (1,)), ((), ())),
                            preferred_element_type=jnp.float32) + bu_ref[...]
        h = g * jax.nn.sigmoid(g) * u
        out_ref[...] = lax.dot_general(h, wd_ref[0], (((1,), (1,)), ((), ())),
                                       preferred_element_type=jnp.float32
                                       ) + bd_ref[...]


def _combine_body(w_ref, y0_ref, y1_ref, out_ref):
    w = w_ref[...]
    out_ref[...] = y0_ref[...] * w[:, 0:1] + y1_ref[...] * w[:, 1:2]


def kernel(x, Wg_router, W_gate, W_up, W_down, b_gate, b_up, b_down):
    xf = x.reshape(_N, _D)

    pos, w, te, nact, bl = pl.pallas_call(
        _router_body,
        out_shape=[
            jax.ShapeDtypeStruct((_N, 2), jnp.int32),
            jax.ShapeDtypeStruct((_N, 2), jnp.float32),
            jax.ShapeDtypeStruct((1, _T), jnp.int32),
            jax.ShapeDtypeStruct((1, 1), jnp.int32),
            jax.ShapeDtypeStruct((1, 1), jnp.float32),
        ],
    )(xf, Wg_router)

    pos_flat = pos.T.reshape(-1)

    xg = _sc_scatter(xf, pos_flat)

    grid_spec = pltpu.PrefetchScalarGridSpec(
        num_scalar_prefetch=2,
        grid=(_T,),
        in_specs=[
            pl.BlockSpec((_BN, _D), lambda t, te, na: (t, 0)),
            pl.BlockSpec((1, _H, _D), lambda t, te, na: (te[0, t], 0, 0)),
            pl.BlockSpec((1, _H, _D), lambda t, te, na: (te[0, t], 0, 0)),
            pl.BlockSpec((1, _D, _H), lambda t, te, na: (te[0, t], 0, 0)),
            pl.BlockSpec((1, _H), lambda t, te, na: (te[0, t], 0)),
            pl.BlockSpec((1, _H), lambda t, te, na: (te[0, t], 0)),
            pl.BlockSpec((1, _D), lambda t, te, na: (te[0, t], 0)),
        ],
        out_specs=pl.BlockSpec((_BN, _D), lambda t, te, na: (t, 0)),
    )
    yg = pl.pallas_call(
        _expert_body,
        grid_spec=grid_spec,
        out_shape=jax.ShapeDtypeStruct((_R, _D), jnp.float32),
    )(te, nact, xg, W_gate, W_up, W_down, b_gate, b_up, b_down)

    yc = _sc_gather(yg, pos_flat)

    nb = _N // _BN
    y = pl.pallas_call(
        _combine_body,
        grid=(nb,),
        in_specs=[
            pl.BlockSpec((_BN, 2), lambda i: (i, 0)),
            pl.BlockSpec((_BN, _D), lambda i: (i, 0)),
            pl.BlockSpec((_BN, _D), lambda i: (i + nb, 0)),
        ],
        out_specs=pl.BlockSpec((_BN, _D), lambda i: (i, 0)),
        out_shape=jax.ShapeDtypeStruct((_N, _D), jnp.float32),
    )(w, yc, yc)

    return y.reshape(x.shape), bl.reshape(())


# speedup vs baseline: 1.5782x; 1.5782x over previous
import functools

import jax
import jax.numpy as jnp
from jax import lax
from jax.experimental import pallas as pl
from jax.experimental.pallas import tpu as pltpu
from jax.experimental.pallas import tpu_sc as plsc

_N = 2048
_D = 1024
_H = 2048
_E = 8
_BN = 128
_T = 40
_R = _T * _BN
_PAIRS = 2 * _N
_NW = 32
_CH = 64
_NCH = _PAIRS // (_NW * _CH)


def _router_body(x_ref, wg_ref, pos_ref, w_ref, te_ref, nact_ref, bl_ref):
    xf = x_ref[...]
    wg = wg_ref[...]
    logits = lax.dot_general(xf, wg, (((1,), (0,)), ((), ())),
                             preferred_element_type=jnp.float32)

    u8 = (lax.broadcasted_iota(jnp.int32, (_E, _E), 0)
          < lax.broadcasted_iota(jnp.int32, (_E, _E), 1)).astype(jnp.float32)

    def first_max_mask(v):
        m = jnp.max(v, axis=1, keepdims=True)
        ism = (v == m).astype(jnp.float32)
        prev = lax.dot_general(ism, u8, (((1,), (0,)), ((), ())),
                               preferred_element_type=jnp.float32)
        return (ism > 0) & (prev == 0.0), m

    oh1b, m1 = first_max_mask(logits)
    masked = jnp.where(oh1b, -1e30, logits)
    oh2b, m2 = first_max_mask(masked)
    s1 = jax.nn.sigmoid(m1 - m2)
    s2 = jax.nn.sigmoid(m2 - m1)
    oh1 = oh1b.astype(jnp.float32)
    oh2 = oh2b.astype(jnp.float32)

    imp = jnp.sum(oh1 * s1 + oh2 * s2, axis=0, keepdims=True)
    load = jnp.sum(oh1 + oh2, axis=0, keepdims=True)

    def cv2(v):
        mean = jnp.sum(v) / _E
        var = jnp.sum((v - mean) ** 2) / (_E - 1)
        return var / (mean * mean + 1e-10)

    bl_ref[...] = jnp.reshape(cv2(imp) * cv2(load) * 0.01, (1, 1))

    ls = (lax.broadcasted_iota(jnp.int32, (_BN, _BN), 0)
          > lax.broadcasted_iota(jnp.int32, (_BN, _BN), 1)).astype(jnp.float32)

    def excl_cumsum(oh):
        chunks = []
        carry = jnp.zeros((1, _E), jnp.float32)
        for c in range(_N // _BN):
            blk = oh[c * _BN:(c + 1) * _BN, :]
            cex = lax.dot_general(ls, blk, (((1,), (0,)), ((), ())),
                                  preferred_element_type=jnp.float32) + carry
            chunks.append(cex)
            carry = carry + jnp.sum(blk, axis=0, keepdims=True)
        return jnp.concatenate(chunks, axis=0), carry

    c1, tot1 = excl_cumsum(oh1)
    c2, tot2 = excl_cumsum(oh2)
    counts = tot1 + tot2
    pcq = (counts.astype(jnp.int32) + (_BN - 1)) // _BN
    baseq = lax.dot_general(pcq.astype(jnp.float32), u8,
                            (((1,), (0,)), ((), ())),
                            preferred_element_type=jnp.float32).astype(jnp.int32)
    ends = baseq + pcq
    nact_ref[...] = jnp.reshape(jnp.sum(pcq), (1, 1))

    t_iota = lax.broadcasted_iota(jnp.int32, (_T, _E), 0)
    te_raw = jnp.sum((t_iota >= jnp.broadcast_to(ends, (_T, _E)))
                     .astype(jnp.int32), axis=1)
    max_e = jnp.max(jnp.where(pcq > 0,
                              lax.broadcasted_iota(jnp.int32, (1, _E), 1), -1))
    te_ref[...] = jnp.minimum(te_raw, max_e).reshape(1, _T)

    basef = (baseq * _BN).astype(jnp.float32)
    base_b = jnp.broadcast_to(basef, (_N, _E))
    rank0 = jnp.sum(c1 * oh1, axis=1, keepdims=True)
    rank1 = jnp.sum((jnp.broadcast_to(tot1, (_N, _E)) + c2) * oh2,
                    axis=1, keepdims=True)
    base0 = jnp.sum(base_b * oh1, axis=1, keepdims=True)
    base1 = jnp.sum(base_b * oh2, axis=1, keepdims=True)
    pos0 = (base0 + rank0).astype(jnp.int32)
    pos1 = (base1 + rank1).astype(jnp.int32)
    pos_ref[...] = jnp.concatenate([pos0, pos1], axis=1)
    w_ref[...] = jnp.concatenate([s1, s2], axis=1)


def _sc_mesh():
    return plsc.VectorSubcoreMesh(core_axis_name="c", subcore_axis_name="s")


def _sc_scatter(x, pos_flat):
    @functools.partial(
        pl.kernel, mesh=_sc_mesh(),
        out_type=jax.ShapeDtypeStruct((_R, _D), jnp.float32),
        scratch_types=[pltpu.VMEM((_CH,), jnp.int32),
                       pltpu.VMEM((_CH, _D), jnp.float32),
                       pltpu.SemaphoreType.DMA])
    def body(x_hbm, pos_hbm, xg_hbm, idx_v, rows_v, sem):
        wid = lax.axis_index("s") * 2 + lax.axis_index("c")
        for c in range(_NCH):
            base = wid * (_CH * _NCH) + c * _CH
            pltpu.sync_copy(pos_hbm.at[pl.ds(base, _CH)], idx_v)
            src = lax.rem(base, _N)
            pltpu.sync_copy(x_hbm.at[pl.ds(src, _CH)], rows_v)
            pltpu.async_copy(rows_v, xg_hbm.at[idx_v], sem).wait()

    return body(x, pos_flat)


def _sc_gather(yg, pos_flat):
    @functools.partial(
        pl.kernel, mesh=_sc_mesh(),
        out_type=jax.ShapeDtypeStruct((_PAIRS, _D), jnp.float32),
        scratch_types=[pltpu.VMEM((_CH,), jnp.int32),
                       pltpu.VMEM((_CH, _D), jnp.float32),
                       pltpu.SemaphoreType.DMA])
    def body(yg_hbm, pos_hbm, yc_hbm, idx_v, rows_v, sem):
        wid = lax.axis_index("s") * 2 + lax.axis_index("c")
        for c in range(_NCH):
            base = wid * (_CH * _NCH) + c * _CH
            pltpu.sync_copy(pos_hbm.at[pl.ds(base, _CH)], idx_v)
            pltpu.async_copy(yg_hbm.at[idx_v], rows_v, sem).wait()
            pltpu.sync_copy(rows_v, yc_hbm.at[pl.ds(base, _CH)])

    return body(yg, pos_flat)


def _expert_body(te_ref, na_ref, xg_ref, wg_ref, wu_ref, wd_ref,
                 bg_ref, bu_ref, bd_ref, out_ref):
    t = pl.program_id(0)

    @pl.when(t < na_ref[0, 0])
    def _():
        xb = xg_ref[...]
        g = lax.dot_general(xb, wg_ref[0], (((1,), (1,)), ((), ())),
                            preferred_element_type=jnp.float32) + bg_ref[0]
        u = lax.dot_general(xb, wu_ref[0], (((1,), (1,)), ((), ())),
                            preferred_element_type=jnp.float32) + bu_ref[0]
        h = g * jax.nn.sigmoid(g) * u
        out_ref[...] = lax.dot_general(h, wd_ref[0], (((1,), (1,)), ((), ())),
                                       preferred_element_type=jnp.float32
                                       ) + bd_ref[0]


def _combine_body(w_ref, y0_ref, y1_ref, out_ref):
    w = w_ref[...]
    out_ref[...] = y0_ref[...] * w[:, 0:1] + y1_ref[...] * w[:, 1:2]


def kernel(x, Wg_router, W_gate, W_up, W_down, b_gate, b_up, b_down):
    xf = x.reshape(_N, _D)

    pos, w, te, nact, bl = pl.pallas_call(
        _router_body,
        out_shape=[
            jax.ShapeDtypeStruct((_N, 2), jnp.int32),
            jax.ShapeDtypeStruct((_N, 2), jnp.float32),
            jax.ShapeDtypeStruct((1, _T), jnp.int32),
            jax.ShapeDtypeStruct((1, 1), jnp.int32),
            jax.ShapeDtypeStruct((1, 1), jnp.float32),
        ],
    )(xf, Wg_router)

    pos_flat = pos.T.reshape(-1)

    xg = _sc_scatter(xf, pos_flat)

    grid_spec = pltpu.PrefetchScalarGridSpec(
        num_scalar_prefetch=2,
        grid=(_T,),
        in_specs=[
            pl.BlockSpec((_BN, _D), lambda t, te, na: (t, 0)),
            pl.BlockSpec((1, _H, _D), lambda t, te, na: (te[0, t], 0, 0)),
            pl.BlockSpec((1, _H, _D), lambda t, te, na: (te[0, t], 0, 0)),
            pl.BlockSpec((1, _D, _H), lambda t, te, na: (te[0, t], 0, 0)),
            pl.BlockSpec((1, 1, _H), lambda t, te, na: (te[0, t], 0, 0)),
            pl.BlockSpec((1, 1, _H), lambda t, te, na: (te[0, t], 0, 0)),
            pl.BlockSpec((1, 1, _D), lambda t, te, na: (te[0, t], 0, 0)),
        ],
        out_specs=pl.BlockSpec((_BN, _D), lambda t, te, na: (t, 0)),
    )
    yg = pl.pallas_call(
        _expert_body,
        grid_spec=grid_spec,
        out_shape=jax.ShapeDtypeStruct((_R, _D), jnp.float32),
    )(te, nact, xg, W_gate, W_up, W_down,
      b_gate.reshape(_E, 1, _H), b_up.reshape(_E, 1, _H),
      b_down.reshape(_E, 1, _D))

    yc = _sc_gather(yg, pos_flat)

    nb = _N // _BN
    y = pl.pallas_call(
        _combine_body,
        grid=(nb,),
        in_specs=[
            pl.BlockSpec((_BN, 2), lambda i: (i, 0)),
            pl.BlockSpec((_BN, _D), lambda i: (i, 0)),
            pl.BlockSpec((_BN, _D), lambda i: (i + nb, 0)),
        ],
        out_specs=pl.BlockSpec((_BN, _D), lambda i: (i, 0)),
        out_shape=jax.ShapeDtypeStruct((_N, _D), jnp.float32),
    )(w, yc, yc)

    return y.reshape(x.shape), bl.reshape(())


# speedup vs baseline: 1.9950x; 1.2641x over previous
import functools

import jax
import jax.numpy as jnp
from jax import lax
from jax.experimental import pallas as pl
from jax.experimental.pallas import tpu as pltpu
from jax.experimental.pallas import tpu_sc as plsc

_N = 2048
_D = 1024
_H = 2048
_E = 8
_BN = 256
_T = 24
_R = _T * _BN
_PAIRS = 2 * _N
_NW = 32
_CH = 64
_NCH = _PAIRS // (_NW * _CH)


def _router_body(x_ref, wg_ref, pos_ref, w_ref, rb_ref, nt_ref, bl_ref):
    xf = x_ref[...]
    wg = wg_ref[...]
    logits = lax.dot_general(xf, wg, (((1,), (0,)), ((), ())),
                             preferred_element_type=jnp.float32)

    u8 = (lax.broadcasted_iota(jnp.int32, (_E, _E), 0)
          < lax.broadcasted_iota(jnp.int32, (_E, _E), 1)).astype(jnp.float32)

    def first_max_mask(v):
        m = jnp.max(v, axis=1, keepdims=True)
        ism = (v == m).astype(jnp.float32)
        prev = lax.dot_general(ism, u8, (((1,), (0,)), ((), ())),
                               preferred_element_type=jnp.float32)
        return (ism > 0) & (prev == 0.0), m

    oh1b, m1 = first_max_mask(logits)
    masked = jnp.where(oh1b, -1e30, logits)
    oh2b, m2 = first_max_mask(masked)
    s1 = jax.nn.sigmoid(m1 - m2)
    s2 = jax.nn.sigmoid(m2 - m1)
    oh1 = oh1b.astype(jnp.float32)
    oh2 = oh2b.astype(jnp.float32)

    imp = jnp.sum(oh1 * s1 + oh2 * s2, axis=0, keepdims=True)
    load = jnp.sum(oh1 + oh2, axis=0, keepdims=True)

    def cv2(v):
        mean = jnp.sum(v) / _E
        var = jnp.sum((v - mean) ** 2) / (_E - 1)
        return var / (mean * mean + 1e-10)

    bl_ref[...] = jnp.reshape(cv2(imp) * cv2(load) * 0.01, (1, 1))

    ls = (lax.broadcasted_iota(jnp.int32, (_BN, _BN), 0)
          > lax.broadcasted_iota(jnp.int32, (_BN, _BN), 1)).astype(jnp.float32)

    def excl_cumsum(oh):
        chunks = []
        carry = jnp.zeros((1, _E), jnp.float32)
        for c in range(_N // _BN):
            blk = oh[c * _BN:(c + 1) * _BN, :]
            cex = lax.dot_general(ls, blk, (((1,), (0,)), ((), ())),
                                  preferred_element_type=jnp.float32) + carry
            chunks.append(cex)
            carry = carry + jnp.sum(blk, axis=0, keepdims=True)
        return jnp.concatenate(chunks, axis=0), carry

    c1, tot1 = excl_cumsum(oh1)
    c2, tot2 = excl_cumsum(oh2)
    counts = tot1 + tot2
    pcq = (counts.astype(jnp.int32) + (_BN - 1)) // _BN
    baseq = lax.dot_general(pcq.astype(jnp.float32), u8,
                            (((1,), (0,)), ((), ())),
                            preferred_element_type=jnp.float32).astype(jnp.int32)
    rb_ref[...] = baseq * _BN
    nt_ref[...] = pcq

    basef = (baseq * _BN).astype(jnp.float32)
    base_b = jnp.broadcast_to(basef, (_N, _E))
    rank0 = jnp.sum(c1 * oh1, axis=1, keepdims=True)
    rank1 = jnp.sum((jnp.broadcast_to(tot1, (_N, _E)) + c2) * oh2,
                    axis=1, keepdims=True)
    base0 = jnp.sum(base_b * oh1, axis=1, keepdims=True)
    base1 = jnp.sum(base_b * oh2, axis=1, keepdims=True)
    pos0 = (base0 + rank0).astype(jnp.int32)
    pos1 = (base1 + rank1).astype(jnp.int32)
    pos_ref[...] = jnp.concatenate([pos0, pos1], axis=1)
    w_ref[...] = jnp.concatenate([s1, s2], axis=1)


def _sc_mesh():
    return plsc.VectorSubcoreMesh(core_axis_name="c", subcore_axis_name="s")


def _sc_scatter(x, pos_flat):
    @functools.partial(
        pl.kernel, mesh=_sc_mesh(),
        out_type=jax.ShapeDtypeStruct((_R, _D), jnp.float32),
        scratch_types=[pltpu.VMEM((_CH,), jnp.int32),
                       pltpu.VMEM((_CH, _D), jnp.float32),
                       pltpu.SemaphoreType.DMA])
    def body(x_hbm, pos_hbm, xg_hbm, idx_v, rows_v, sem):
        wid = lax.axis_index("s") * 2 + lax.axis_index("c")
        for c in range(_NCH):
            base = wid * (_CH * _NCH) + c * _CH
            pltpu.sync_copy(pos_hbm.at[pl.ds(base, _CH)], idx_v)
            src = lax.rem(base, _N)
            pltpu.sync_copy(x_hbm.at[pl.ds(src, _CH)], rows_v)
            pltpu.async_copy(rows_v, xg_hbm.at[idx_v], sem).wait()

    return body(x, pos_flat)


def _sc_gather(yg, pos_flat):
    @functools.partial(
        pl.kernel, mesh=_sc_mesh(),
        out_type=jax.ShapeDtypeStruct((_PAIRS, _D), jnp.float32),
        scratch_types=[pltpu.VMEM((_CH,), jnp.int32),
                       pltpu.VMEM((_CH, _D), jnp.float32),
                       pltpu.SemaphoreType.DMA])
    def body(yg_hbm, pos_hbm, yc_hbm, idx_v, rows_v, sem):
        wid = lax.axis_index("s") * 2 + lax.axis_index("c")
        for c in range(_NCH):
            base = wid * (_CH * _NCH) + c * _CH
            pltpu.sync_copy(pos_hbm.at[pl.ds(base, _CH)], idx_v)
            pltpu.async_copy(yg_hbm.at[idx_v], rows_v, sem).wait()
            pltpu.sync_copy(rows_v, yc_hbm.at[pl.ds(base, _CH)])

    return body(yg, pos_flat)


def _expert_body(rb_ref, nt_ref, xg_hbm, wg_hbm, wu_hbm, wd_hbm,
                 bg_ref, bu_ref, bd_ref, yg_hbm, xbuf, ybuf,
                 wgbuf, wubuf, wdbuf, sin, sout, swt):
    e = pl.program_id(0)
    nt = nt_ref[0, e]
    row0 = rb_ref[0, e]

    def w_copies(eidx, sel):
        return (pltpu.make_async_copy(wg_hbm.at[eidx], wgbuf.at[sel],
                                      swt.at[sel, 0]),
                pltpu.make_async_copy(wu_hbm.at[eidx], wubuf.at[sel],
                                      swt.at[sel, 1]),
                pltpu.make_async_copy(wd_hbm.at[eidx], wdbuf.at[sel],
                                      swt.at[sel, 2]))

    wsel = lax.rem(e, 2)

    @pl.when(e == 0)
    def _():
        for c in w_copies(0, 0):
            c.start()

    @pl.when(e + 1 < _E)
    def _():
        for c in w_copies(e + 1, 1 - wsel):
            c.start()

    for c in w_copies(e, wsel):
        c.wait()

    def cp_in(j, sel):
        start = pl.multiple_of(row0 + j * _BN, _BN)
        return pltpu.make_async_copy(
            xg_hbm.at[pl.ds(start, _BN), :], xbuf.at[sel],
            sin.at[sel])

    def cp_out(j, sel):
        start = pl.multiple_of(row0 + j * _BN, _BN)
        return pltpu.make_async_copy(
            ybuf.at[sel], yg_hbm.at[pl.ds(start, _BN), :],
            sout.at[sel])

    @pl.when(nt > 0)
    def _():
        cp_in(0, 0).start()

    def tile(j, carry):
        sel = lax.rem(j, 2)

        @pl.when(j + 1 < nt)
        def _():
            cp_in(j + 1, 1 - sel).start()

        cp_in(j, sel).wait()
        xb = xbuf[sel]
        g = lax.dot_general(xb, wgbuf[wsel], (((1,), (1,)), ((), ())),
                            preferred_element_type=jnp.float32) + bg_ref[0]
        u = lax.dot_general(xb, wubuf[wsel], (((1,), (1,)), ((), ())),
                            preferred_element_type=jnp.float32) + bu_ref[0]
        h = g * jax.nn.sigmoid(g) * u
        o = lax.dot_general(h, wdbuf[wsel], (((1,), (1,)), ((), ())),
                            preferred_element_type=jnp.float32) + bd_ref[0]

        @pl.when(j >= 2)
        def _():
            cp_out(j - 2, sel).wait()

        ybuf[sel] = o
        cp_out(j, sel).start()
        return carry

    lax.fori_loop(0, nt, tile, 0)

    @pl.when(nt >= 2)
    def _():
        cp_out(nt - 2, lax.rem(nt - 2, 2)).wait()

    @pl.when(nt >= 1)
    def _():
        cp_out(nt - 1, lax.rem(nt - 1, 2)).wait()


def _combine_body(w_ref, y0_ref, y1_ref, out_ref):
    w = w_ref[...]
    out_ref[...] = y0_ref[...] * w[:, 0:1] + y1_ref[...] * w[:, 1:2]


def kernel(x, Wg_router, W_gate, W_up, W_down, b_gate, b_up, b_down):
    xf = x.reshape(_N, _D)

    pos, w, rowbase, ntiles, bl = pl.pallas_call(
        _router_body,
        out_shape=[
            jax.ShapeDtypeStruct((_N, 2), jnp.int32),
            jax.ShapeDtypeStruct((_N, 2), jnp.float32),
            jax.ShapeDtypeStruct((1, _E), jnp.int32),
            jax.ShapeDtypeStruct((1, _E), jnp.int32),
            jax.ShapeDtypeStruct((1, 1), jnp.float32),
        ],
    )(xf, Wg_router)

    pos_flat = pos.T.reshape(-1)

    xg = _sc_scatter(xf, pos_flat)

    grid_spec = pltpu.PrefetchScalarGridSpec(
        num_scalar_prefetch=2,
        grid=(_E,),
        in_specs=[
            pl.BlockSpec(memory_space=pl.ANY),
            pl.BlockSpec(memory_space=pl.ANY),
            pl.BlockSpec(memory_space=pl.ANY),
            pl.BlockSpec(memory_space=pl.ANY),
            pl.BlockSpec((1, 1, _H), lambda e, rb, nt: (e, 0, 0)),
            pl.BlockSpec((1, 1, _H), lambda e, rb, nt: (e, 0, 0)),
            pl.BlockSpec((1, 1, _D), lambda e, rb, nt: (e, 0, 0)),
        ],
        out_specs=pl.BlockSpec(memory_space=pl.ANY),
        scratch_shapes=[
            pltpu.VMEM((2, _BN, _D), jnp.float32),
            pltpu.VMEM((2, _BN, _D), jnp.float32),
            pltpu.VMEM((2, _H, _D), jnp.float32),
            pltpu.VMEM((2, _H, _D), jnp.float32),
            pltpu.VMEM((2, _D, _H), jnp.float32),
            pltpu.SemaphoreType.DMA((2,)),
            pltpu.SemaphoreType.DMA((2,)),
            pltpu.SemaphoreType.DMA((2, 3)),
        ],
    )
    yg = pl.pallas_call(
        _expert_body,
        grid_spec=grid_spec,
        out_shape=jax.ShapeDtypeStruct((_R, _D), jnp.float32),
    )(rowbase, ntiles, xg, W_gate, W_up, W_down,
      b_gate.reshape(_E, 1, _H), b_up.reshape(_E, 1, _H),
      b_down.reshape(_E, 1, _D))

    yc = _sc_gather(yg, pos_flat)

    nb = _N // _BN
    y = pl.pallas_call(
        _combine_body,
        grid=(nb,),
        in_specs=[
            pl.BlockSpec((_BN, 2), lambda i: (i, 0)),
            pl.BlockSpec((_BN, _D), lambda i: (i, 0)),
            pl.BlockSpec((_BN, _D), lambda i: (i + nb, 0)),
        ],
        out_specs=pl.BlockSpec((_BN, _D), lambda i: (i, 0)),
        out_shape=jax.ShapeDtypeStruct((_N, _D), jnp.float32),
    )(w, yc, yc)

    return y.reshape(x.shape), bl.reshape(())
